# Initial kernel scaffold; baseline (speedup 1.0000x reference)
import jax
import jax.numpy as jnp
from jax.experimental import pallas as pl


def kernel(feature_u, feature_v, side_feature_u, side_feature_v, W, W_su, b_su, W_sv, b_sv, W_cat_u, W_cat_v, Q, edge_u_dst, edge_u_src):
    raise NotImplementedError("write your pallas kernel here")


# speedup vs baseline: 3.8957x; 3.8957x over previous
import functools

import jax
import jax.numpy as jnp
from jax import lax
from jax.experimental import pallas as pl
from jax.experimental.pallas import tpu as pltpu
from jax.experimental.pallas import tpu_sc as plsc

F32 = jnp.float32
NC, NS, CH, TRASH = 2, 16, 128, 128


def _pad_to(x, m):
    return (x + m - 1) // m * m


def _dims(n_u, n_v):
    half = _pad_to((n_u + 1) // 2, 8)
    own = _pad_to(half, 8 * NS)
    n_vp = _pad_to(n_v, 8 * NS)
    return half, own, n_vp


def _make_pre(n_u, n_v):
    def pre_body(fu, fv, su, sv, w0, wsu, bsu, wsv, bsv,
                 msg_u, msg_v, shu, shv):
        msg_u[0:n_u, :] = jnp.dot(fu[...], w0[...],
                                  preferred_element_type=F32)
        msg_v[0:n_v, :] = jnp.dot(fv[...], w0[...],
                                  preferred_element_type=F32)
        dn = (((1,), (1,)), ((), ()))
        shu[...] = jnp.maximum(
            lax.dot_general(su[...], wsu[...], dn,
                            preferred_element_type=F32) + bsu[...], 0.0)
        shv[...] = jnp.maximum(
            lax.dot_general(sv[...], wsv[...], dn,
                            preferred_element_type=F32) + bsv[...], 0.0)
    return pre_body


def _make_sc_agg(n_u, n_v, e, h):
    half, own, n_vp = _dims(n_u, n_v)
    acc_r = own + TRASH
    ept = e // NS
    nfull = ept // CH
    tail = ept - nfull * CH
    assert tail % 16 == 0 and ept % 8 == 0
    rz = acc_r // NS
    r_own = own // NS
    r_v = n_vp // NS

    mesh = plsc.VectorSubcoreMesh(core_axis_name="c", subcore_axis_name="s")

    scratch = [
        pltpu.VMEM((CH,), jnp.int32),
        pltpu.VMEM((CH,), jnp.int32),
        pltpu.VMEM((CH,), jnp.int32),
        pltpu.VMEM((CH, h), F32),
        pltpu.VMEM((rz, h), F32),
        pltpu.VMEM_SHARED((acc_r, h), F32),
        pltpu.VMEM_SHARED((n_vp, h), F32),
        pltpu.SemaphoreType.DMA,
    ]
    if tail:
        scratch += [
            pltpu.VMEM((tail,), jnp.int32),
            pltpu.VMEM((tail,), jnp.int32),
            pltpu.VMEM((tail,), jnp.int32),
            pltpu.VMEM((tail, h), F32),
        ]

    @functools.partial(
        pl.kernel,
        out_type=[jax.ShapeDtypeStruct((NC * own, h), F32),
                  jax.ShapeDtypeStruct((NC * n_vp, h), F32)],
        mesh=mesh,
        scratch_types=scratch,
    )
    def sc_agg(msgu_hbm, msgv_hbm, src_hbm, dst_hbm, out_u, out_v,
               idx_s, idx_d, idx_m, rows, zbuf, sp_big, sp_small, sem,
               *tail_bufs):
        c = lax.axis_index("c")
        s = lax.axis_index("s")
        lo = c * half
        base = s * ept
        lane = lax.iota(jnp.int32, 16)

        def zrow(r, carry):
            for k in range(h // 16):
                zbuf[r, pl.ds(k * 16, 16)] = jnp.zeros((16,), F32)
            return carry
        lax.fori_loop(0, rz, zrow, 0)

        def remap(dref, mref, n, j):
            for k in range(n // 16):
                dv = dref[pl.ds(k * 16, 16)]
                ok = (dv >= lo) & (dv < lo + half)
                alt = own + jnp.bitwise_and(j + k, 7) * 16 + lane
                mref[pl.ds(k * 16, 16)] = jnp.where(ok, dv - lo, alt)

        pltpu.sync_copy(msgv_hbm.at[pl.ds(s * r_v, r_v)],
                        sp_small.at[pl.ds(s * r_v, r_v)])
        pltpu.sync_copy(zbuf, sp_big.at[pl.ds(s * rz, rz)])
        plsc.subcore_barrier()

        def chunk_u(j, carry):
            off = base + j * CH
            pltpu.sync_copy(src_hbm.at[pl.ds(off, CH)], idx_s)
            pltpu.sync_copy(dst_hbm.at[pl.ds(off, CH)], idx_d)
            remap(idx_d, idx_m, CH, j)
            pltpu.async_copy(sp_small.at[idx_s], rows, sem).wait()
            pltpu.sync_copy(rows, sp_big.at[idx_m], add=True)
            return carry
        lax.fori_loop(0, nfull, chunk_u, 0)
        if tail:
            tidx_s, tidx_d, tidx_m, trows = tail_bufs
            off = base + nfull * CH
            pltpu.sync_copy(src_hbm.at[pl.ds(off, tail)], tidx_s)
            pltpu.sync_copy(dst_hbm.at[pl.ds(off, tail)], tidx_d)
            remap(tidx_d, tidx_m, tail, nfull)
            pltpu.async_copy(sp_small.at[tidx_s], trows, sem).wait()
            pltpu.sync_copy(trows, sp_big.at[tidx_m], add=True)
        plsc.subcore_barrier()
        pltpu.sync_copy(sp_big.at[pl.ds(s * r_own, r_own)],
                        out_u.at[pl.ds(c * own + s * r_own, r_own)])
        plsc.subcore_barrier()

        pltpu.sync_copy(msgu_hbm.at[pl.ds(lo + s * r_own, r_own)],
                        sp_big.at[pl.ds(s * r_own, r_own)])

        @pl.when(s == 0)
        def _zero_trash():
            pltpu.sync_copy(zbuf.at[pl.ds(0, TRASH)],
                            sp_big.at[pl.ds(own, TRASH)])

        pltpu.sync_copy(zbuf.at[pl.ds(0, r_v)],
                        sp_small.at[pl.ds(s * r_v, r_v)])
        plsc.subcore_barrier()

        def chunk_v(j, carry):
            off = base + j * CH
            pltpu.sync_copy(src_hbm.at[pl.ds(off, CH)], idx_s)
            pltpu.sync_copy(dst_hbm.at[pl.ds(off, CH)], idx_d)
            remap(idx_d, idx_m, CH, j)
            pltpu.async_copy(sp_big.at[idx_m], rows, sem).wait()
            pltpu.sync_copy(rows, sp_small.at[idx_s], add=True)
            return carry
        lax.fori_loop(0, nfull, chunk_v, 0)
        if tail:
            tidx_s, tidx_d, tidx_m, trows = tail_bufs
            off = base + nfull * CH
            pltpu.sync_copy(src_hbm.at[pl.ds(off, tail)], tidx_s)
            pltpu.sync_copy(dst_hbm.at[pl.ds(off, tail)], tidx_d)
            remap(tidx_d, tidx_m, tail, nfull)
            pltpu.async_copy(sp_big.at[tidx_m], trows, sem).wait()
            pltpu.sync_copy(trows, sp_small.at[tidx_s], add=True)
        plsc.subcore_barrier()
        pltpu.sync_copy(sp_small.at[pl.ds(s * r_v, r_v)],
                        out_v.at[pl.ds(c * n_vp + s * r_v, r_v)])

    return sc_agg


def _make_post(n_u, n_v, h):
    half, own, n_vp = _dims(n_u, n_v)

    def post_body(aggu, aggv, shu, shv, wcu, wcv, q0, amat, ev):
        dn = (((1,), (1,)), ((), ()))
        hid_u = jnp.concatenate(
            [aggu[0:half, :], aggu[own:own + (n_u - half), :]], axis=0)
        hu = jnp.maximum(hid_u, 0.0)
        hv = jnp.maximum(aggv[0:n_v, :] + aggv[n_vp:n_vp + n_v, :], 0.0)
        wcu1, wcu2 = wcu[:, 0:h], wcu[:, h:]
        wcv1, wcv2 = wcv[:, 0:h], wcv[:, h:]
        eu = jnp.maximum(
            lax.dot_general(hu, wcu1, dn, preferred_element_type=F32)
            + lax.dot_general(shu[...], wcu2, dn,
                              preferred_element_type=F32), 0.0)
        ev[...] = jnp.maximum(
            lax.dot_general(hv, wcv1, dn, preferred_element_type=F32)
            + lax.dot_general(shv[...], wcv2, dn,
                              preferred_element_type=F32), 0.0)
        amat[...] = jnp.dot(eu, q0[...], preferred_element_type=F32)
    return post_body


def _score_body(ev, amat, out):
    dn = (((1,), (1,)), ((), ()))
    out[...] = lax.dot_general(ev[...], amat[...], dn,
                               preferred_element_type=F32)


def kernel(feature_u, feature_v, side_feature_u, side_feature_v,
           W, W_su, b_su, W_sv, b_sv, W_cat_u, W_cat_v, Q,
           edge_u_dst, edge_u_src):
    n_u, d = feature_u.shape
    n_v = feature_v.shape[0]
    h = W.shape[2]
    sh = W_su.shape[0]
    out_dim = W_cat_u.shape[0]
    e = edge_u_dst.shape[0]
    half, own, n_vp = _dims(n_u, n_v)

    w0 = W[0]
    q0 = Q[0]
    bsu = b_su.reshape(1, sh)
    bsv = b_sv.reshape(1, sh)

    msg_u, msg_v, shu, shv = pl.pallas_call(
        _make_pre(n_u, n_v),
        out_shape=[
            jax.ShapeDtypeStruct((half + own, h), F32),
            jax.ShapeDtypeStruct((n_vp, h), F32),
            jax.ShapeDtypeStruct((n_u, sh), F32),
            jax.ShapeDtypeStruct((n_v, sh), F32),
        ],
    )(feature_u, feature_v, side_feature_u, side_feature_v,
      w0, W_su, bsu, W_sv, bsv)

    aggu, aggv = _make_sc_agg(n_u, n_v, e, h)(
        msg_u, msg_v, edge_u_src, edge_u_dst)

    amat, ev = pl.pallas_call(
        _make_post(n_u, n_v, h),
        out_shape=[
            jax.ShapeDtypeStruct((n_u, out_dim), F32),
            jax.ShapeDtypeStruct((n_v, out_dim), F32),
        ],
    )(aggu, aggv, shu, shv, W_cat_u, W_cat_v, q0)

    bm = 1024
    score_t = pl.pallas_call(
        _score_body,
        grid=((n_u + bm - 1) // bm,),
        in_specs=[
            pl.BlockSpec((n_v, out_dim), lambda i: (0, 0)),
            pl.BlockSpec((bm, out_dim), lambda i: (i, 0)),
        ],
        out_specs=pl.BlockSpec((n_v, bm), lambda i: (0, i)),
        out_shape=jax.ShapeDtypeStruct((n_v, n_u), F32),
    )(ev, amat)

    return score_t.T[None]


# speedup vs baseline: 7.5572x; 1.9399x over previous
import functools

import jax
import jax.numpy as jnp
from jax import lax
from jax.experimental import pallas as pl
from jax.experimental.pallas import tpu as pltpu
from jax.experimental.pallas import tpu_sc as plsc

F32 = jnp.float32
NC, NS, CH, TRASH = 2, 16, 128, 128


def _pad_to(x, m):
    return (x + m - 1) // m * m


def _dims(n_u, n_v):
    half = _pad_to((n_u + 1) // 2, 8)
    own = _pad_to(half, 8 * NS)
    n_vp = _pad_to(n_v, 8 * NS)
    return half, own, n_vp


def _make_pre(n_u, n_v):
    def pre_body(fu, fv, su, sv, w0, wsu, bsu, wsv, bsv,
                 msg_u, msg_v, shu, shv):
        msg_u[0:n_u, :] = jnp.dot(fu[...], w0[...],
                                  preferred_element_type=F32)
        msg_v[0:n_v, :] = jnp.dot(fv[...], w0[...],
                                  preferred_element_type=F32)
        dn = (((1,), (1,)), ((), ()))
        shu[...] = jnp.maximum(
            lax.dot_general(su[...], wsu[...], dn,
                            preferred_element_type=F32) + bsu[...], 0.0)
        shv[...] = jnp.maximum(
            lax.dot_general(sv[...], wsv[...], dn,
                            preferred_element_type=F32) + bsv[...], 0.0)
    return pre_body


def _make_sc_agg(n_u, n_v, e, h):
    half, own, n_vp = _dims(n_u, n_v)
    acc_r = own + TRASH
    ept = e // NS
    nfull = ept // CH
    tail = ept - nfull * CH
    assert tail % 16 == 0 and ept % 8 == 0
    rz = acc_r // NS
    r_own = own // NS
    r_v = n_vp // NS
    ZB = 8
    assert rz % ZB == 0 and r_v % ZB == 0 and TRASH % ZB == 0

    mesh = plsc.VectorSubcoreMesh(core_axis_name="c", subcore_axis_name="s")

    NBUF = 4
    scratch = (
        [pltpu.VMEM((CH,), jnp.int32) for _ in range(NBUF)]
        + [pltpu.VMEM((CH,), jnp.int32) for _ in range(NBUF)]
        + [pltpu.VMEM((CH,), jnp.int32) for _ in range(NBUF)]
        + [pltpu.VMEM((CH, h), F32) for _ in range(NBUF)]
        + [pltpu.VMEM((ZB, h), F32)]
        + [pltpu.VMEM_SHARED((acc_r, h), F32),
           pltpu.VMEM_SHARED((n_vp, h), F32)]
        + [pltpu.SemaphoreType.DMA for _ in range(4 * NBUF + 1)]
    )
    if tail:
        scratch += [
            pltpu.VMEM((tail,), jnp.int32),
            pltpu.VMEM((tail, h), F32),
        ]

    @functools.partial(
        pl.kernel,
        out_type=[jax.ShapeDtypeStruct((NC * own, h), F32),
                  jax.ShapeDtypeStruct((NC * n_vp, h), F32)],
        mesh=mesh,
        scratch_types=scratch,
    )
    def sc_agg(msgu_hbm, msgv_hbm, src_hbm, dst_hbm, out_u, out_v,
               *refs):
        ics = refs[0:NBUF]
        icd = refs[NBUF:2 * NBUF]
        im = refs[2 * NBUF:3 * NBUF]
        rows = refs[3 * NBUF:4 * NBUF]
        zbuf, sp_big, sp_small = refs[4 * NBUF:3 + 4 * NBUF]
        sems = refs[3 + 4 * NBUF:4 + 8 * NBUF]
        isem, dsem = sems[0:NBUF], sems[NBUF:2 * NBUF]
        gs, ss = sems[2 * NBUF:3 * NBUF], sems[3 * NBUF:4 * NBUF]
        sem1 = sems[4 * NBUF]
        tail_bufs = refs[4 + 8 * NBUF:]

        c = lax.axis_index("c")
        s = lax.axis_index("s")
        lo = c * half
        base = s * ept
        lane = lax.iota(jnp.int32, 16)
        ngrp = nfull // NBUF

        def zrow(r, carry):
            for k in range(h // 16):
                zbuf[r, pl.ds(k * 16, 16)] = jnp.zeros((16,), F32)
            return carry
        lax.fori_loop(0, ZB, zrow, 0)

        def zero_spmem(ref, r0, n):
            def zcp(t, carry):
                pltpu.sync_copy(zbuf, ref.at[pl.ds(r0 + t * ZB, ZB)])
                return carry
            lax.fori_loop(0, n // ZB, zcp, 0)

        def remap(dref, mref, n, salt):
            for k in range(n // 16):
                dv = dref[pl.ds(k * 16, 16)]
                ok = (dv >= lo) & (dv < lo + half)
                alt = own + jnp.bitwise_and(salt + k, 7) * 16 + lane
                mref[pl.ds(k * 16, 16)] = jnp.where(ok, dv - lo, alt)

        pltpu.sync_copy(msgv_hbm.at[pl.ds(s * r_v, r_v)],
                        sp_small.at[pl.ds(s * r_v, r_v)])
        zero_spmem(sp_big, s * rz, rz)
        plsc.subcore_barrier()

        def grp_u(g, carry):
            jb = g * NBUF
            di = []
            for q in range(NBUF):
                off = base + (jb + q) * CH
                di.append((
                    pltpu.async_copy(src_hbm.at[pl.ds(off, CH)],
                                     ics[q], isem[q]),
                    pltpu.async_copy(dst_hbm.at[pl.ds(off, CH)],
                                     icd[q], dsem[q])))
            gd = []
            for q in range(NBUF):
                di[q][0].wait()
                di[q][1].wait()
                remap(icd[q], im[q], CH, jb + q)
                gd.append(pltpu.async_copy(
                    sp_small.at[ics[q]], rows[q], gs[q]))
            sd = []
            for q in range(NBUF):
                gd[q].wait()
                sd.append(pltpu.async_copy(
                    rows[q], sp_big.at[im[q]], ss[q], add=True))
            for q in range(NBUF):
                sd[q].wait()
            return carry
        lax.fori_loop(0, ngrp, grp_u, 0)
        for j in range(ngrp * NBUF, nfull):
            off = base + j * CH
            pltpu.sync_copy(src_hbm.at[pl.ds(off, CH)], ics[0])
            pltpu.sync_copy(dst_hbm.at[pl.ds(off, CH)], icd[0])
            remap(icd[0], im[0], CH, j)
            pltpu.async_copy(sp_small.at[ics[0]], rows[0], gs[0]).wait()
            pltpu.sync_copy(rows[0], sp_big.at[im[0]], add=True)
        if tail:
            tidx, trows = tail_bufs
            toff = base + nfull * CH
            pltpu.sync_copy(src_hbm.at[pl.ds(toff, tail)], tidx)
            pltpu.async_copy(sp_small.at[tidx], trows, sem1).wait()
            pltpu.sync_copy(dst_hbm.at[pl.ds(toff, tail)], tidx)
            remap(tidx, tidx, tail, nfull)
            pltpu.sync_copy(trows, sp_big.at[tidx], add=True)

        plsc.subcore_barrier()
        pltpu.sync_copy(sp_big.at[pl.ds(s * r_own, r_own)],
                        out_u.at[pl.ds(c * own + s * r_own, r_own)])
        plsc.subcore_barrier()

        pltpu.sync_copy(msgu_hbm.at[pl.ds(lo + s * r_own, r_own)],
                        sp_big.at[pl.ds(s * r_own, r_own)])

        @pl.when(s == 0)
        def _zero_trash():
            zero_spmem(sp_big, own, TRASH)

        zero_spmem(sp_small, s * r_v, r_v)
        plsc.subcore_barrier()

        def grp_v(g, carry):
            jb = g * NBUF
            di = []
            for q in range(NBUF):
                off = base + (jb + q) * CH
                di.append((
                    pltpu.async_copy(src_hbm.at[pl.ds(off, CH)],
                                     ics[q], isem[q]),
                    pltpu.async_copy(dst_hbm.at[pl.ds(off, CH)],
                                     icd[q], dsem[q])))
            gd = []
            for q in range(NBUF):
                di[q][0].wait()
                di[q][1].wait()
                remap(icd[q], im[q], CH, jb + q)
                gd.append(pltpu.async_copy(
                    sp_big.at[im[q]], rows[q], gs[q]))
            sd = []
            for q in range(NBUF):
                gd[q].wait()
                sd.append(pltpu.async_copy(
                    rows[q], sp_small.at[ics[q]], ss[q], add=True))
            for q in range(NBUF):
                sd[q].wait()
            return carry
        lax.fori_loop(0, ngrp, grp_v, 0)
        for j in range(ngrp * NBUF, nfull):
            off = base + j * CH
            pltpu.sync_copy(src_hbm.at[pl.ds(off, CH)], ics[0])
            pltpu.sync_copy(dst_hbm.at[pl.ds(off, CH)], icd[0])
            remap(icd[0], im[0], CH, j)
            pltpu.async_copy(sp_big.at[im[0]], rows[0], gs[0]).wait()
            pltpu.sync_copy(rows[0], sp_small.at[ics[0]], add=True)
        if tail:
            tidx, trows = tail_bufs
            toff = base + nfull * CH
            pltpu.sync_copy(dst_hbm.at[pl.ds(toff, tail)], tidx)
            remap(tidx, tidx, tail, nfull)
            pltpu.async_copy(sp_big.at[tidx], trows, sem1).wait()
            pltpu.sync_copy(src_hbm.at[pl.ds(toff, tail)], tidx)
            pltpu.sync_copy(trows, sp_small.at[tidx], add=True)
        plsc.subcore_barrier()
        pltpu.sync_copy(sp_small.at[pl.ds(s * r_v, r_v)],
                        out_v.at[pl.ds(c * n_vp + s * r_v, r_v)])

    return sc_agg


def _make_post(n_u, n_v, h):
    half, own, n_vp = _dims(n_u, n_v)

    def post_body(aggu, aggv, shu, shv, wcu, wcv, q0, amat, ev):
        dn = (((1,), (1,)), ((), ()))
        hid_u = jnp.concatenate(
            [aggu[0:half, :], aggu[own:own + (n_u - half), :]], axis=0)
        hu = jnp.maximum(hid_u, 0.0)
        hv = jnp.maximum(aggv[0:n_v, :] + aggv[n_vp:n_vp + n_v, :], 0.0)
        wcu1, wcu2 = wcu[:, 0:h], wcu[:, h:]
        wcv1, wcv2 = wcv[:, 0:h], wcv[:, h:]
        eu = jnp.maximum(
            lax.dot_general(hu, wcu1, dn, preferred_element_type=F32)
            + lax.dot_general(shu[...], wcu2, dn,
                              preferred_element_type=F32), 0.0)
        ev[...] = jnp.maximum(
            lax.dot_general(hv, wcv1, dn, preferred_element_type=F32)
            + lax.dot_general(shv[...], wcv2, dn,
                              preferred_element_type=F32), 0.0)
        amat[...] = jnp.dot(eu, q0[...], preferred_element_type=F32)
    return post_body


def _score_body(ev, amat, out):
    dn = (((1,), (1,)), ((), ()))
    out[...] = lax.dot_general(ev[...], amat[...], dn,
                               preferred_element_type=F32)


def kernel(feature_u, feature_v, side_feature_u, side_feature_v,
           W, W_su, b_su, W_sv, b_sv, W_cat_u, W_cat_v, Q,
           edge_u_dst, edge_u_src):
    n_u, d = feature_u.shape
    n_v = feature_v.shape[0]
    h = W.shape[2]
    sh = W_su.shape[0]
    out_dim = W_cat_u.shape[0]
    e = edge_u_dst.shape[0]
    half, own, n_vp = _dims(n_u, n_v)

    w0 = W[0]
    q0 = Q[0]
    bsu = b_su.reshape(1, sh)
    bsv = b_sv.reshape(1, sh)

    msg_u, msg_v, shu, shv = pl.pallas_call(
        _make_pre(n_u, n_v),
        out_shape=[
            jax.ShapeDtypeStruct((half + own, h), F32),
            jax.ShapeDtypeStruct((n_vp, h), F32),
            jax.ShapeDtypeStruct((n_u, sh), F32),
            jax.ShapeDtypeStruct((n_v, sh), F32),
        ],
    )(feature_u, feature_v, side_feature_u, side_feature_v,
      w0, W_su, bsu, W_sv, bsv)

    aggu, aggv = _make_sc_agg(n_u, n_v, e, h)(
        msg_u, msg_v, edge_u_src, edge_u_dst)

    amat, ev = pl.pallas_call(
        _make_post(n_u, n_v, h),
        out_shape=[
            jax.ShapeDtypeStruct((n_u, out_dim), F32),
            jax.ShapeDtypeStruct((n_v, out_dim), F32),
        ],
    )(aggu, aggv, shu, shv, W_cat_u, W_cat_v, q0)

    bm = 1024
    score_t = pl.pallas_call(
        _score_body,
        grid=((n_u + bm - 1) // bm,),
        in_specs=[
            pl.BlockSpec((n_v, out_dim), lambda i: (0, 0)),
            pl.BlockSpec((bm, out_dim), lambda i: (i, 0)),
        ],
        out_specs=pl.BlockSpec((n_v, bm), lambda i: (0, i)),
        out_shape=jax.ShapeDtypeStruct((n_v, n_u), F32),
    )(ev, amat)

    return score_t.T[None]
